# baseline (speedup 1.0000x reference)
import functools
import math

import jax
import jax.numpy as jnp
from jax import lax
from jax.experimental import pallas as pl
from jax.experimental.pallas import tpu as pltpu
from jax.experimental.pallas import tpu_sc as plsc

SEQ = 50
BATCH = 16384
EMB = 64
SCALE = math.sqrt(EMB)

NC = 2
NS = 16
NW = NC * NS

B_PER_W = BATCH // NW
CHUNK = 128
NCHUNK = B_PER_W // CHUNK
NBUF = 4

_mesh = plsc.VectorSubcoreMesh(core_axis_name="c", subcore_axis_name="s")


@functools.partial(
    pl.kernel,
    out_type=jax.ShapeDtypeStruct((SEQ * BATCH, EMB), jnp.float32),
    mesh=_mesh,
    compiler_params=pltpu.CompilerParams(use_tc_tiling_on_sc=False),
    scratch_types=(
        [pltpu.VMEM((SEQ, B_PER_W), jnp.int32)]
        + [pltpu.VMEM((CHUNK, EMB), jnp.float32) for _ in range(NBUF)]
        + [pltpu.VMEM((CHUNK, EMB), jnp.float32) for _ in range(NBUF)]
        + [pltpu.VMEM((SEQ, EMB), jnp.float32)]
        + [pltpu.SemaphoreType.DMA]
        + [pltpu.SemaphoreType.DMA for _ in range(NBUF)]
        + [pltpu.SemaphoreType.DMA for _ in range(NBUF)]
    ),
)
def _emb_kernel(data_hbm, table_hbm, pos_hbm, out_hbm,
                idx_all, g0, g1, g2, g3, o0, o1, o2, o3, pos_v,
                idx_sem, gs0, gs1, gs2, gs3, os0, os1, os2, os3):
    gbuf = [g0, g1, g2, g3]
    obuf = [o0, o1, o2, o3]
    gsem = [gs0, gs1, gs2, gs3]
    osem = [os0, os1, os2, os3]

    wid = lax.axis_index("s") * NC + lax.axis_index("c")
    col0 = wid * B_PER_W

    idx_cp = pltpu.make_async_copy(
        data_hbm.at[:, pl.ds(col0, B_PER_W)], idx_all, idx_sem)
    idx_cp.start()
    pltpu.sync_copy(pos_hbm, pos_v)
    idx_cp.wait()

    for b in range(NBUF):
        pltpu.make_async_copy(
            table_hbm.at[idx_all.at[0, pl.ds(CHUNK * b, CHUNK)]],
            gbuf[b], gsem[b]).start()

    def outer(o, carry):
        pvec = [pos_v[o, pl.ds(16 * jj, 16)] for jj in range(4)]
        row_base = o * BATCH + col0

        for b in range(NBUF):
            pltpu.make_async_copy(
                table_hbm.at[idx_all.at[o, pl.ds(CHUNK * b, CHUNK)]],
                gbuf[b], gsem[b]).wait()

            @pl.when(o > 0)
            def _():
                pltpu.make_async_copy(
                    obuf[b], out_hbm.at[pl.ds(0, CHUNK)], osem[b]).wait()

            gb = gbuf[b]
            ob = obuf[b]

            def row_body(r, rc, gb=gb, ob=ob, pvec=pvec):
                for jj in range(4):
                    v = gb[r, pl.ds(16 * jj, 16)]
                    ob[r, pl.ds(16 * jj, 16)] = v * SCALE + pvec[jj]
                return rc

            lax.fori_loop(0, CHUNK, row_body, 0, unroll=2)

            @pl.when(o < SEQ - 1)
            def _():
                pltpu.make_async_copy(
                    table_hbm.at[idx_all.at[o + 1, pl.ds(CHUNK * b, CHUNK)]],
                    gbuf[b], gsem[b]).start()

            pltpu.make_async_copy(
                ob, out_hbm.at[pl.ds(row_base + CHUNK * b, CHUNK)],
                osem[b]).start()
        return carry

    lax.fori_loop(0, SEQ, outer, 0)

    for b in range(NBUF):
        pltpu.make_async_copy(
            obuf[b], out_hbm.at[pl.ds(0, CHUNK)], osem[b]).wait()


def kernel(data, token_table, position_table):
    out = _emb_kernel(data.astype(jnp.int32), token_table, position_table)
    return out.reshape(SEQ, BATCH, EMB)
